# baseline (speedup 1.0000x reference)
import functools

import jax
import jax.numpy as jnp
from jax import lax
from jax.experimental import pallas as pl
from jax.experimental.pallas import tpu as pltpu
from jax.experimental.pallas import tpu_sc as plsc

N_NODES = 10000
N_EDGES = 320000
D = 128

NC = 2
NS = 16
NW = NC * NS

B = 128
NB = 80
E_PAD = NW * NB * B
CH = 632
N_PAD = NS * CH
DUMMY_DST = N_NODES + 8


def _mm_body(x_ref, w_ref, b_ref, o_ref):
    o_ref[...] = (
        jnp.dot(x_ref[...], w_ref[...], preferred_element_type=jnp.float32)
        + b_ref[...]
    )


def _matmul(x, W, b2d):
    m_blk = 1000
    return pl.pallas_call(
        _mm_body,
        grid=(N_NODES // m_blk,),
        in_specs=[
            pl.BlockSpec((m_blk, D), lambda i: (i, 0)),
            pl.BlockSpec((D, D), lambda i: (0, 0)),
            pl.BlockSpec((1, D), lambda i: (0, 0)),
        ],
        out_specs=pl.BlockSpec((m_blk, D), lambda i: (i, 0)),
        out_shape=jax.ShapeDtypeStruct((N_NODES, D), jnp.float32),
    )(x, W, b2d)


_sc_mesh = plsc.VectorSubcoreMesh(core_axis_name="c", subcore_axis_name="s")


@functools.partial(
    pl.kernel,
    out_type=jax.ShapeDtypeStruct((NC, N_PAD, D), jnp.float32),
    mesh=_sc_mesh,
    scratch_types=[
        pltpu.VMEM((NB, B), jnp.int32),
        pltpu.VMEM((2, B), jnp.int32),
        [pltpu.VMEM((B, D), jnp.float32) for _ in range(2)],
        pltpu.VMEM_SHARED((N_PAD, D), jnp.float32),
        [pltpu.SemaphoreType.DMA for _ in range(2)],
        [pltpu.SemaphoreType.DMA for _ in range(2)],
    ],
)
def _sc_push(h_hbm, srcs_hbm, dsts_hbm, zeros_hbm, out_hbm,
             src_v, dst_ring, rows, acc, rsems, dsems):
    c = lax.axis_index("c")
    s = lax.axis_index("s")
    wid = c * NS + s

    pltpu.sync_copy(srcs_hbm.at[wid], src_v)
    pltpu.sync_copy(zeros_hbm.at[pl.ds(s * CH, CH)], acc.at[pl.ds(s * CH, CH)])
    plsc.subcore_barrier()

    nbuf = 2
    for b in range(nbuf):
        pltpu.async_copy(h_hbm.at[src_v.at[b]], rows[b], rsems[b])
        pltpu.async_copy(dsts_hbm.at[wid, b], dst_ring.at[b], dsems[b])

    @pl.loop(0, NB - nbuf, step=nbuf)
    def _batch(g):
        for b in range(nbuf):
            j = g + b
            pltpu.make_async_copy(h_hbm.at[src_v.at[j]], rows[b],
                                  rsems[b]).wait()
            pltpu.make_async_copy(dsts_hbm.at[wid, j], dst_ring.at[b],
                                  dsems[b]).wait()
            pltpu.sync_copy(rows[b], acc.at[dst_ring.at[b]], add=True)
            pltpu.async_copy(h_hbm.at[src_v.at[j + nbuf]], rows[b], rsems[b])
            pltpu.async_copy(dsts_hbm.at[wid, j + nbuf], dst_ring.at[b],
                             dsems[b])

    for b in range(nbuf):
        j = NB - nbuf + b
        pltpu.make_async_copy(h_hbm.at[src_v.at[j]], rows[b], rsems[b]).wait()
        pltpu.make_async_copy(dsts_hbm.at[wid, j], dst_ring.at[b],
                              dsems[b]).wait()
        pltpu.sync_copy(rows[b], acc.at[dst_ring.at[b]], add=True)

    plsc.subcore_barrier()
    pltpu.sync_copy(acc.at[pl.ds(s * CH, CH)],
                    out_hbm.at[c, pl.ds(s * CH, CH)])


def _comb_body(a_ref, b_ref, o_ref):
    o_ref[...] = jnp.maximum(a_ref[...] + b_ref[...], 0.0)


def _combine(a, b):
    m_blk = 1000
    return pl.pallas_call(
        _comb_body,
        grid=(N_NODES // m_blk,),
        in_specs=[
            pl.BlockSpec((m_blk, D), lambda i: (i, 0)),
            pl.BlockSpec((m_blk, D), lambda i: (i, 0)),
        ],
        out_specs=pl.BlockSpec((m_blk, D), lambda i: (i, 0)),
        out_shape=jax.ShapeDtypeStruct((N_NODES, D), jnp.float32),
    )(a, b)


@jax.jit
def kernel(x, edge_index, W, b):
    h = _matmul(x, W, b.reshape(1, D))

    src = edge_index[0].astype(jnp.int32)
    dst = edge_index[1].astype(jnp.int32)
    pad = E_PAD - N_EDGES
    src = jnp.concatenate([src, jnp.zeros((pad,), jnp.int32)])
    dst = jnp.concatenate([dst, jnp.full((pad,), DUMMY_DST, jnp.int32)])
    srcs = src.reshape(NW, NB, B)
    dsts = dst.reshape(NW, NB, B)
    zeros = jnp.zeros((N_PAD, D), jnp.float32)

    acc = _sc_push(h, srcs, dsts, zeros)
    return _combine(acc[0, :N_NODES], acc[1, :N_NODES])


# speedup vs baseline: 1.7840x; 1.7840x over previous
import functools

import jax
import jax.numpy as jnp
from jax import lax
from jax.experimental import pallas as pl
from jax.experimental.pallas import tpu as pltpu
from jax.experimental.pallas import tpu_sc as plsc

N_NODES = 10000
N_EDGES = 320000
D = 128

NC = 2
NS = 16
NW = NC * NS

B = 128
NB0 = 126
NB1 = 32
E0 = NS * NB0 * B
E1_PAD = NS * NB1 * B
CH = 632
N_PAD = NS * CH
DUMMY_DST = N_NODES + 8


def _mm_body(x_ref, w_ref, b_ref, o_ref):
    o_ref[...] = (
        jnp.dot(x_ref[...], w_ref[...], preferred_element_type=jnp.float32)
        + b_ref[...]
    )


def _matmul(x, W, b2d):
    m_blk = 1000
    return pl.pallas_call(
        _mm_body,
        grid=(N_NODES // m_blk,),
        in_specs=[
            pl.BlockSpec((m_blk, D), lambda i: (i, 0)),
            pl.BlockSpec((D, D), lambda i: (0, 0)),
            pl.BlockSpec((1, D), lambda i: (0, 0)),
        ],
        out_specs=pl.BlockSpec((m_blk, D), lambda i: (i, 0)),
        out_shape=jax.ShapeDtypeStruct((N_NODES, D), jnp.float32),
    )(x, W, b2d)


_sc_mesh = plsc.VectorSubcoreMesh(core_axis_name="c", subcore_axis_name="s")


@functools.partial(
    pl.kernel,
    out_type=jax.ShapeDtypeStruct((NC, N_PAD, D), jnp.float32),
    mesh=_sc_mesh,
    scratch_types=[
        pltpu.VMEM((NB0, B), jnp.int32),
        pltpu.VMEM((2, B), jnp.int32),
        [pltpu.VMEM((B, D), jnp.float32) for _ in range(2)],
        pltpu.VMEM_SHARED((N_PAD, D), jnp.float32),
        [pltpu.SemaphoreType.DMA for _ in range(2)],
        [pltpu.SemaphoreType.DMA for _ in range(2)],
    ],
)
def _sc_push(h_hbm, srcs_hbm, dsts_hbm, zeros_hbm, out_hbm,
             src_v, dst_ring, rows, acc, rsems, dsems):
    c = lax.axis_index("c")
    s = lax.axis_index("s")
    wid = c * NS + s

    pltpu.sync_copy(srcs_hbm.at[wid], src_v)
    pltpu.sync_copy(zeros_hbm.at[pl.ds(s * CH, CH)], acc.at[pl.ds(s * CH, CH)])
    plsc.subcore_barrier()

    nbuf = 2
    nb_c = jnp.where(c == 0, NB0, NB1)
    for b in range(nbuf):
        pltpu.async_copy(h_hbm.at[src_v.at[b]], rows[b], rsems[b])
        pltpu.async_copy(dsts_hbm.at[wid, b], dst_ring.at[b], dsems[b])

    @pl.loop(0, nb_c - nbuf, step=nbuf)
    def _batch(g):
        for b in range(nbuf):
            j = g + b
            pltpu.make_async_copy(h_hbm.at[src_v.at[j]], rows[b],
                                  rsems[b]).wait()
            pltpu.make_async_copy(dsts_hbm.at[wid, j], dst_ring.at[b],
                                  dsems[b]).wait()
            pltpu.sync_copy(rows[b], acc.at[dst_ring.at[b]], add=True)
            pltpu.async_copy(h_hbm.at[src_v.at[j + nbuf]], rows[b], rsems[b])
            pltpu.async_copy(dsts_hbm.at[wid, j + nbuf], dst_ring.at[b],
                             dsems[b])

    for b in range(nbuf):
        j = nb_c - nbuf + b
        pltpu.make_async_copy(h_hbm.at[src_v.at[j]], rows[b], rsems[b]).wait()
        pltpu.make_async_copy(dsts_hbm.at[wid, j], dst_ring.at[b],
                              dsems[b]).wait()
        pltpu.sync_copy(rows[b], acc.at[dst_ring.at[b]], add=True)

    plsc.subcore_barrier()
    pltpu.sync_copy(acc.at[pl.ds(s * CH, CH)],
                    out_hbm.at[c, pl.ds(s * CH, CH)])


def _comb_body(a_ref, b_ref, o_ref):
    o_ref[...] = jnp.maximum(a_ref[...] + b_ref[...], 0.0)


def _combine(a, b):
    m_blk = 1000
    return pl.pallas_call(
        _comb_body,
        grid=(N_NODES // m_blk,),
        in_specs=[
            pl.BlockSpec((m_blk, D), lambda i: (i, 0)),
            pl.BlockSpec((m_blk, D), lambda i: (i, 0)),
        ],
        out_specs=pl.BlockSpec((m_blk, D), lambda i: (i, 0)),
        out_shape=jax.ShapeDtypeStruct((N_NODES, D), jnp.float32),
    )(a, b)


@jax.jit
def kernel(x, edge_index, W, b):
    h = _matmul(x, W, b.reshape(1, D))

    src = edge_index[0].astype(jnp.int32)
    dst = edge_index[1].astype(jnp.int32)
    pad = E1_PAD - (N_EDGES - E0)
    src1 = jnp.concatenate([src[E0:], jnp.zeros((pad,), jnp.int32)])
    dst1 = jnp.concatenate([dst[E0:], jnp.full((pad,), DUMMY_DST, jnp.int32)])
    srcs = jnp.concatenate([
        src[:E0].reshape(NS, NB0, B),
        jnp.pad(src1.reshape(NS, NB1, B), ((0, 0), (0, NB0 - NB1), (0, 0))),
    ])
    dsts = jnp.concatenate([
        dst[:E0].reshape(NS, NB0, B),
        jnp.pad(dst1.reshape(NS, NB1, B), ((0, 0), (0, NB0 - NB1), (0, 0)),
                constant_values=DUMMY_DST),
    ])
    zeros = jnp.zeros((N_PAD, D), jnp.float32)

    acc = _sc_push(h, srcs, dsts, zeros)
    return _combine(acc[0, :N_NODES], acc[1, :N_NODES])


# speedup vs baseline: 1.8142x; 1.0169x over previous
import functools

import jax
import jax.numpy as jnp
from jax import lax
from jax.experimental import pallas as pl
from jax.experimental.pallas import tpu as pltpu
from jax.experimental.pallas import tpu_sc as plsc

N_NODES = 10000
N_EDGES = 320000
D = 128

NC = 2
NS = 16
NW = NC * NS

B = 128
NB0 = 134
NB1 = 24
E0 = NS * NB0 * B
E1_PAD = NS * NB1 * B
CH = 624
TAIL = 24
N_PAD = NS * CH + TAIL
DUMMY_DST = N_NODES


def _mm_body(x_ref, w_ref, b_ref, o_ref):
    o_ref[...] = (
        jnp.dot(x_ref[...], w_ref[...], preferred_element_type=jnp.float32)
        + b_ref[...]
    )


def _matmul(x, W, b2d):
    m_blk = 1000
    return pl.pallas_call(
        _mm_body,
        grid=(N_NODES // m_blk,),
        in_specs=[
            pl.BlockSpec((m_blk, D), lambda i: (i, 0)),
            pl.BlockSpec((D, D), lambda i: (0, 0)),
            pl.BlockSpec((1, D), lambda i: (0, 0)),
        ],
        out_specs=pl.BlockSpec((m_blk, D), lambda i: (i, 0)),
        out_shape=jax.ShapeDtypeStruct((N_NODES, D), jnp.float32),
    )(x, W, b2d)


_sc_mesh = plsc.VectorSubcoreMesh(core_axis_name="c", subcore_axis_name="s")


@functools.partial(
    pl.kernel,
    out_type=jax.ShapeDtypeStruct((NC, N_PAD, D), jnp.float32),
    mesh=_sc_mesh,
    scratch_types=[
        pltpu.VMEM((NB0, B), jnp.int32),
        pltpu.VMEM((2, B), jnp.int32),
        [pltpu.VMEM((B, D), jnp.float32) for _ in range(2)],
        pltpu.VMEM_SHARED((N_PAD, D), jnp.float32),
        [pltpu.SemaphoreType.DMA for _ in range(2)],
        [pltpu.SemaphoreType.DMA for _ in range(2)],
    ],
)
def _sc_push(h_hbm, srcs_hbm, dsts_hbm, out_hbm,
             src_v, dst_ring, rows, acc, rsems, dsems):
    c = lax.axis_index("c")
    s = lax.axis_index("s")
    wid = c * NS + s

    pltpu.sync_copy(srcs_hbm.at[wid], src_v)
    zero16 = jnp.zeros((16,), jnp.float32)
    for i in range(B):
        for k in range(D // 16):
            rows[0][i, pl.ds(k * 16, 16)] = zero16
    base = s * CH
    for t in range(CH // B):
        pltpu.sync_copy(rows[0], acc.at[pl.ds(base + t * B, B)])
    rem = CH - (CH // B) * B
    pltpu.sync_copy(rows[0].at[pl.ds(0, rem)],
                    acc.at[pl.ds(base + (CH // B) * B, rem)])

    @pl.when(s == 0)
    def _ztail():
        pltpu.sync_copy(rows[0].at[pl.ds(0, TAIL)],
                        acc.at[pl.ds(NS * CH, TAIL)])

    plsc.subcore_barrier()

    nbuf = 2
    nb_c = jnp.where(c == 0, NB0, NB1)
    for b in range(nbuf):
        pltpu.async_copy(h_hbm.at[src_v.at[b]], rows[b], rsems[b])
        pltpu.async_copy(dsts_hbm.at[wid, b], dst_ring.at[b], dsems[b])

    @pl.loop(0, nb_c - nbuf, step=nbuf)
    def _batch(g):
        for b in range(nbuf):
            j = g + b
            pltpu.make_async_copy(h_hbm.at[src_v.at[j]], rows[b],
                                  rsems[b]).wait()
            pltpu.make_async_copy(dsts_hbm.at[wid, j], dst_ring.at[b],
                                  dsems[b]).wait()
            pltpu.sync_copy(rows[b], acc.at[dst_ring.at[b]], add=True)
            pltpu.async_copy(h_hbm.at[src_v.at[j + nbuf]], rows[b], rsems[b])
            pltpu.async_copy(dsts_hbm.at[wid, j + nbuf], dst_ring.at[b],
                             dsems[b])

    for b in range(nbuf):
        j = nb_c - nbuf + b
        pltpu.make_async_copy(h_hbm.at[src_v.at[j]], rows[b], rsems[b]).wait()
        pltpu.make_async_copy(dsts_hbm.at[wid, j], dst_ring.at[b],
                              dsems[b]).wait()
        pltpu.sync_copy(rows[b], acc.at[dst_ring.at[b]], add=True)

    plsc.subcore_barrier()
    pltpu.sync_copy(acc.at[pl.ds(s * CH, CH)],
                    out_hbm.at[c, pl.ds(s * CH, CH)])

    @pl.when(s == 0)
    def _wtail():
        pltpu.sync_copy(acc.at[pl.ds(NS * CH, TAIL)],
                        out_hbm.at[c, pl.ds(NS * CH, TAIL)])


def _comb_body(a_ref, b_ref, o_ref):
    o_ref[...] = jnp.maximum(a_ref[...] + b_ref[...], 0.0)


def _combine(a, b):
    m_blk = 1000
    return pl.pallas_call(
        _comb_body,
        grid=(N_NODES // m_blk,),
        in_specs=[
            pl.BlockSpec((m_blk, D), lambda i: (i, 0)),
            pl.BlockSpec((m_blk, D), lambda i: (i, 0)),
        ],
        out_specs=pl.BlockSpec((m_blk, D), lambda i: (i, 0)),
        out_shape=jax.ShapeDtypeStruct((N_NODES, D), jnp.float32),
    )(a, b)


@jax.jit
def kernel(x, edge_index, W, b):
    h = _matmul(x, W, b.reshape(1, D))

    src = edge_index[0].astype(jnp.int32)
    dst = edge_index[1].astype(jnp.int32)
    pad = E1_PAD - (N_EDGES - E0)
    src1 = jnp.concatenate([src[E0:], jnp.zeros((pad,), jnp.int32)])
    dst1 = jnp.concatenate([dst[E0:], jnp.full((pad,), DUMMY_DST, jnp.int32)])
    srcs = jnp.concatenate([
        src[:E0].reshape(NS, NB0, B),
        jnp.pad(src1.reshape(NS, NB1, B), ((0, 0), (0, NB0 - NB1), (0, 0))),
    ])
    dsts = jnp.concatenate([
        dst[:E0].reshape(NS, NB0, B),
        jnp.pad(dst1.reshape(NS, NB1, B), ((0, 0), (0, NB0 - NB1), (0, 0)),
                constant_values=DUMMY_DST),
    ])
    acc = _sc_push(h, srcs, dsts)
    return _combine(acc[0, :N_NODES], acc[1, :N_NODES])


# speedup vs baseline: 1.8696x; 1.0305x over previous
import functools

import jax
import jax.numpy as jnp
from jax import lax
from jax.experimental import pallas as pl
from jax.experimental.pallas import tpu as pltpu
from jax.experimental.pallas import tpu_sc as plsc

N_NODES = 10000
N_EDGES = 320000
D = 128

NC = 2
NS = 16
NW = NC * NS

B = 128
NB0 = 136
NB1 = 22
E0 = NS * NB0 * B
E1_PAD = NS * NB1 * B
CH = 624
TAIL = 24
N_PAD = NS * CH + TAIL
DUMMY_DST = N_NODES


def _mm_body(x_ref, w_ref, b_ref, o_ref):
    o_ref[...] = (
        jnp.dot(x_ref[...], w_ref[...], preferred_element_type=jnp.float32)
        + b_ref[...]
    )


def _matmul(x, W, b2d):
    m_blk = 1000
    return pl.pallas_call(
        _mm_body,
        grid=(N_NODES // m_blk,),
        in_specs=[
            pl.BlockSpec((m_blk, D), lambda i: (i, 0)),
            pl.BlockSpec((D, D), lambda i: (0, 0)),
            pl.BlockSpec((1, D), lambda i: (0, 0)),
        ],
        out_specs=pl.BlockSpec((m_blk, D), lambda i: (i, 0)),
        out_shape=jax.ShapeDtypeStruct((N_NODES, D), jnp.float32),
    )(x, W, b2d)


_sc_mesh = plsc.VectorSubcoreMesh(core_axis_name="c", subcore_axis_name="s")


@functools.partial(
    pl.kernel,
    out_type=jax.ShapeDtypeStruct((NC, N_PAD, D), jnp.float32),
    mesh=_sc_mesh,
    scratch_types=[
        pltpu.VMEM((NB0, B), jnp.int32),
        pltpu.VMEM((2, B), jnp.int32),
        [pltpu.VMEM((B, D), jnp.float32) for _ in range(2)],
        pltpu.VMEM_SHARED((N_PAD, D), jnp.float32),
        [pltpu.SemaphoreType.DMA for _ in range(2)],
        [pltpu.SemaphoreType.DMA for _ in range(2)],
    ],
)
def _sc_push(h_hbm, srcs_hbm, dsts_hbm, out_hbm,
             src_v, dst_ring, rows, acc, rsems, dsems):
    c = lax.axis_index("c")
    s = lax.axis_index("s")
    wid = c * NS + s

    pltpu.sync_copy(srcs_hbm.at[wid], src_v)
    zero16 = jnp.zeros((16,), jnp.float32)
    for i in range(B):
        for k in range(D // 16):
            rows[0][i, pl.ds(k * 16, 16)] = zero16
    base = s * CH
    for t in range(CH // B):
        pltpu.sync_copy(rows[0], acc.at[pl.ds(base + t * B, B)])
    rem = CH - (CH // B) * B
    pltpu.sync_copy(rows[0].at[pl.ds(0, rem)],
                    acc.at[pl.ds(base + (CH // B) * B, rem)])

    @pl.when(s == 0)
    def _ztail():
        pltpu.sync_copy(rows[0].at[pl.ds(0, TAIL)],
                        acc.at[pl.ds(NS * CH, TAIL)])

    plsc.subcore_barrier()

    nbuf = 2
    nb_c = jnp.where(c == 0, NB0, NB1)
    for b in range(nbuf):
        pltpu.async_copy(h_hbm.at[src_v.at[b]], rows[b], rsems[b])
        pltpu.async_copy(dsts_hbm.at[wid, b], dst_ring.at[b], dsems[b])

    @pl.loop(0, nb_c - nbuf, step=nbuf)
    def _batch(g):
        for b in range(nbuf):
            j = g + b
            pltpu.make_async_copy(h_hbm.at[src_v.at[j]], rows[b],
                                  rsems[b]).wait()
            pltpu.make_async_copy(dsts_hbm.at[wid, j], dst_ring.at[b],
                                  dsems[b]).wait()
            pltpu.sync_copy(rows[b], acc.at[dst_ring.at[b]], add=True)
            pltpu.async_copy(h_hbm.at[src_v.at[j + nbuf]], rows[b], rsems[b])
            pltpu.async_copy(dsts_hbm.at[wid, j + nbuf], dst_ring.at[b],
                             dsems[b])

    for b in range(nbuf):
        j = nb_c - nbuf + b
        pltpu.make_async_copy(h_hbm.at[src_v.at[j]], rows[b], rsems[b]).wait()
        pltpu.make_async_copy(dsts_hbm.at[wid, j], dst_ring.at[b],
                              dsems[b]).wait()
        pltpu.sync_copy(rows[b], acc.at[dst_ring.at[b]], add=True)

    plsc.subcore_barrier()
    pltpu.sync_copy(acc.at[pl.ds(s * CH, CH)],
                    out_hbm.at[c, pl.ds(s * CH, CH)])

    @pl.when(s == 0)
    def _wtail():
        pltpu.sync_copy(acc.at[pl.ds(NS * CH, TAIL)],
                        out_hbm.at[c, pl.ds(NS * CH, TAIL)])


def _comb_body(a_ref, b_ref, o_ref):
    o_ref[...] = jnp.maximum(a_ref[...] + b_ref[...], 0.0)


def _combine(a, b):
    m_blk = 1000
    return pl.pallas_call(
        _comb_body,
        grid=(N_NODES // m_blk,),
        in_specs=[
            pl.BlockSpec((m_blk, D), lambda i: (i, 0)),
            pl.BlockSpec((m_blk, D), lambda i: (i, 0)),
        ],
        out_specs=pl.BlockSpec((m_blk, D), lambda i: (i, 0)),
        out_shape=jax.ShapeDtypeStruct((N_NODES, D), jnp.float32),
    )(a, b)


@jax.jit
def kernel(x, edge_index, W, b):
    h = _matmul(x, W, b.reshape(1, D))

    src = edge_index[0].astype(jnp.int32)
    dst = edge_index[1].astype(jnp.int32)
    pad = E1_PAD - (N_EDGES - E0)
    src1 = jnp.concatenate([src[E0:], jnp.zeros((pad,), jnp.int32)])
    dst1 = jnp.concatenate([dst[E0:], jnp.full((pad,), DUMMY_DST, jnp.int32)])
    srcs = jnp.concatenate([
        src[:E0].reshape(NS, NB0, B),
        jnp.pad(src1.reshape(NS, NB1, B), ((0, 0), (0, NB0 - NB1), (0, 0))),
    ])
    dsts = jnp.concatenate([
        dst[:E0].reshape(NS, NB0, B),
        jnp.pad(dst1.reshape(NS, NB1, B), ((0, 0), (0, NB0 - NB1), (0, 0)),
                constant_values=DUMMY_DST),
    ])
    acc = _sc_push(h, srcs, dsts)
    return _combine(acc[0, :N_NODES], acc[1, :N_NODES])


# speedup vs baseline: 1.8853x; 1.0084x over previous
import functools

import jax
import jax.numpy as jnp
from jax import lax
from jax.experimental import pallas as pl
from jax.experimental.pallas import tpu as pltpu
from jax.experimental.pallas import tpu_sc as plsc

N_NODES = 10000
N_EDGES = 320000
D = 128

NC = 2
NS = 16
NW = NC * NS

B = 128
NB0 = 136
NB1 = 22
E0 = NS * NB0 * B
E1_PAD = NS * NB1 * B
CH = 624
TAIL = 24
N_PAD = NS * CH + TAIL
DUMMY_DST = N_NODES


def _mm_body(x_ref, w_ref, b_ref, o_ref):
    o_ref[...] = (
        jnp.dot(x_ref[...], w_ref[...], preferred_element_type=jnp.float32)
        + b_ref[...]
    )


def _matmul(x, W, b2d):
    m_blk = 1000
    return pl.pallas_call(
        _mm_body,
        grid=(N_NODES // m_blk,),
        in_specs=[
            pl.BlockSpec((m_blk, D), lambda i: (i, 0)),
            pl.BlockSpec((D, D), lambda i: (0, 0)),
            pl.BlockSpec((1, D), lambda i: (0, 0)),
        ],
        out_specs=pl.BlockSpec((m_blk, D), lambda i: (i, 0)),
        out_shape=jax.ShapeDtypeStruct((N_NODES, D), jnp.float32),
    )(x, W, b2d)


_sc_mesh = plsc.VectorSubcoreMesh(core_axis_name="c", subcore_axis_name="s")


def _edge_pipeline(h_hbm, srcs_hbm, dsts_hbm, src_v, dst_ring, rows, acc,
                   rsems, dsems, s, nb):
    nbuf = 2
    pltpu.sync_copy(srcs_hbm.at[s], src_v.at[pl.ds(0, nb)])
    for b in range(nbuf):
        pltpu.async_copy(h_hbm.at[src_v.at[b]], rows[b], rsems[b])
        pltpu.async_copy(dsts_hbm.at[s, b], dst_ring.at[b], dsems[b])

    @pl.loop(0, nb - nbuf, step=nbuf)
    def _batch(g):
        for b in range(nbuf):
            j = g + b
            pltpu.make_async_copy(h_hbm.at[src_v.at[j]], rows[b],
                                  rsems[b]).wait()
            pltpu.make_async_copy(dsts_hbm.at[s, j], dst_ring.at[b],
                                  dsems[b]).wait()
            pltpu.sync_copy(rows[b], acc.at[dst_ring.at[b]], add=True)
            pltpu.async_copy(h_hbm.at[src_v.at[j + nbuf]], rows[b], rsems[b])
            pltpu.async_copy(dsts_hbm.at[s, j + nbuf], dst_ring.at[b],
                             dsems[b])

    for b in range(nbuf):
        j = nb - nbuf + b
        pltpu.make_async_copy(h_hbm.at[src_v.at[j]], rows[b], rsems[b]).wait()
        pltpu.make_async_copy(dsts_hbm.at[s, j], dst_ring.at[b],
                              dsems[b]).wait()
        pltpu.sync_copy(rows[b], acc.at[dst_ring.at[b]], add=True)


@functools.partial(
    pl.kernel,
    out_type=jax.ShapeDtypeStruct((NC, N_PAD, D), jnp.float32),
    mesh=_sc_mesh,
    scratch_types=[
        pltpu.VMEM((NB0, B), jnp.int32),
        pltpu.VMEM((2, B), jnp.int32),
        [pltpu.VMEM((B, D), jnp.float32) for _ in range(2)],
        pltpu.VMEM_SHARED((N_PAD, D), jnp.float32),
        [pltpu.SemaphoreType.DMA for _ in range(2)],
        [pltpu.SemaphoreType.DMA for _ in range(2)],
    ],
)
def _sc_push(h_hbm, srcs0_hbm, dsts0_hbm, srcs1_hbm, dsts1_hbm, out_hbm,
             src_v, dst_ring, rows, acc, rsems, dsems):
    c = lax.axis_index("c")
    s = lax.axis_index("s")

    zero16 = jnp.zeros((16,), jnp.float32)
    for i in range(B):
        for k in range(D // 16):
            rows[0][i, pl.ds(k * 16, 16)] = zero16
    base = s * CH
    for t in range(CH // B):
        pltpu.sync_copy(rows[0], acc.at[pl.ds(base + t * B, B)])
    rem = CH - (CH // B) * B
    pltpu.sync_copy(rows[0].at[pl.ds(0, rem)],
                    acc.at[pl.ds(base + (CH // B) * B, rem)])

    @pl.when(s == 0)
    def _ztail():
        pltpu.sync_copy(rows[0].at[pl.ds(0, TAIL)],
                        acc.at[pl.ds(NS * CH, TAIL)])

    plsc.subcore_barrier()

    @pl.when(c == 0)
    def _sc0():
        _edge_pipeline(h_hbm, srcs0_hbm, dsts0_hbm, src_v, dst_ring, rows,
                       acc, rsems, dsems, s, NB0)

    @pl.when(c != 0)
    def _sc1():
        _edge_pipeline(h_hbm, srcs1_hbm, dsts1_hbm, src_v, dst_ring, rows,
                       acc, rsems, dsems, s, NB1)

    plsc.subcore_barrier()
    pltpu.sync_copy(acc.at[pl.ds(s * CH, CH)],
                    out_hbm.at[c, pl.ds(s * CH, CH)])

    @pl.when(s == 0)
    def _wtail():
        pltpu.sync_copy(acc.at[pl.ds(NS * CH, TAIL)],
                        out_hbm.at[c, pl.ds(NS * CH, TAIL)])


def _comb_body(a_ref, b_ref, o_ref):
    o_ref[...] = jnp.maximum(a_ref[...] + b_ref[...], 0.0)


def _combine(a, b):
    m_blk = 1000
    return pl.pallas_call(
        _comb_body,
        grid=(N_NODES // m_blk,),
        in_specs=[
            pl.BlockSpec((m_blk, D), lambda i: (i, 0)),
            pl.BlockSpec((m_blk, D), lambda i: (i, 0)),
        ],
        out_specs=pl.BlockSpec((m_blk, D), lambda i: (i, 0)),
        out_shape=jax.ShapeDtypeStruct((N_NODES, D), jnp.float32),
    )(a, b)


@jax.jit
def kernel(x, edge_index, W, b):
    h = _matmul(x, W, b.reshape(1, D))

    src = edge_index[0].astype(jnp.int32)
    dst = edge_index[1].astype(jnp.int32)
    pad = E1_PAD - (N_EDGES - E0)
    srcs0 = src[:E0].reshape(NS, NB0, B)
    dsts0 = dst[:E0].reshape(NS, NB0, B)
    srcs1 = jnp.concatenate(
        [src[E0:], jnp.zeros((pad,), jnp.int32)]).reshape(NS, NB1, B)
    dsts1 = jnp.concatenate(
        [dst[E0:], jnp.full((pad,), DUMMY_DST, jnp.int32)]).reshape(NS, NB1, B)
    acc = _sc_push(h, srcs0, dsts0, srcs1, dsts1)
    return _combine(acc[0, :N_NODES], acc[1, :N_NODES])


# speedup vs baseline: 1.9621x; 1.0407x over previous
import functools

import jax
import jax.numpy as jnp
from jax import lax
from jax.experimental import pallas as pl
from jax.experimental.pallas import tpu as pltpu
from jax.experimental.pallas import tpu_sc as plsc

N_NODES = 10000
N_EDGES = 320000
D = 128

NC = 2
NS = 16
NW = NC * NS

B = 128
NB0 = 136
NB1 = 22
E0 = NS * NB0 * B
E1_PAD = NS * NB1 * B
CH = 624
TAIL = 24
N_PAD = NS * CH + TAIL
DUMMY_DST = N_NODES


def _mm_body(x_ref, w_ref, b_ref, o_ref):
    o_ref[...] = (
        jnp.dot(x_ref[...], w_ref[...], preferred_element_type=jnp.float32)
        + b_ref[...]
    )


def _matmul(x, W, b2d):
    m_blk = 1000
    return pl.pallas_call(
        _mm_body,
        grid=(N_NODES // m_blk,),
        in_specs=[
            pl.BlockSpec((m_blk, D), lambda i: (i, 0)),
            pl.BlockSpec((D, D), lambda i: (0, 0)),
            pl.BlockSpec((1, D), lambda i: (0, 0)),
        ],
        out_specs=pl.BlockSpec((m_blk, D), lambda i: (i, 0)),
        out_shape=jax.ShapeDtypeStruct((N_NODES, D), jnp.float32),
    )(x, W, b2d)


_sc_mesh = plsc.VectorSubcoreMesh(core_axis_name="c", subcore_axis_name="s")


def _edge_pipeline(h_hbm, srcs_hbm, dsts_hbm, src_v, dst_ring, rows, acc,
                   rsems, dsems, s, nb):
    nbuf = 2
    pltpu.sync_copy(srcs_hbm.at[s], src_v.at[pl.ds(0, nb)])
    for b in range(nbuf):
        pltpu.async_copy(h_hbm.at[src_v.at[b]], rows[b], rsems[b])
        pltpu.async_copy(dsts_hbm.at[s, b], dst_ring.at[b], dsems[b])

    @pl.loop(0, nb - nbuf, step=nbuf)
    def _batch(g):
        for b in range(nbuf):
            j = g + b
            pltpu.make_async_copy(h_hbm.at[src_v.at[j]], rows[b],
                                  rsems[b]).wait()
            pltpu.make_async_copy(dsts_hbm.at[s, j], dst_ring.at[b],
                                  dsems[b]).wait()
            pltpu.sync_copy(rows[b], acc.at[dst_ring.at[b]], add=True)
            pltpu.async_copy(h_hbm.at[src_v.at[j + nbuf]], rows[b], rsems[b])
            pltpu.async_copy(dsts_hbm.at[s, j + nbuf], dst_ring.at[b],
                             dsems[b])

    for b in range(nbuf):
        j = nb - nbuf + b
        pltpu.make_async_copy(h_hbm.at[src_v.at[j]], rows[b], rsems[b]).wait()
        pltpu.make_async_copy(dsts_hbm.at[s, j], dst_ring.at[b],
                              dsems[b]).wait()
        pltpu.sync_copy(rows[b], acc.at[dst_ring.at[b]], add=True)


@functools.partial(
    pl.kernel,
    out_type=jax.ShapeDtypeStruct((NC, N_PAD, D), jnp.float32),
    mesh=_sc_mesh,
    scratch_types=[
        pltpu.VMEM((NB0, B), jnp.int32),
        pltpu.VMEM((2, B), jnp.int32),
        [pltpu.VMEM((B, D), jnp.float32) for _ in range(2)],
        pltpu.VMEM_SHARED((N_PAD, D), jnp.float32),
        [pltpu.SemaphoreType.DMA for _ in range(2)],
        [pltpu.SemaphoreType.DMA for _ in range(2)],
    ],
)
def _sc_push(h_hbm, srcs0_hbm, dsts0_hbm, srcs1_hbm, dsts1_hbm, out_hbm,
             src_v, dst_ring, rows, acc, rsems, dsems):
    c = lax.axis_index("c")
    s = lax.axis_index("s")

    zero16 = jnp.zeros((16,), jnp.float32)
    for i in range(B):
        for k in range(D // 16):
            rows[0][i, pl.ds(k * 16, 16)] = zero16
    base = s * CH
    for t in range(CH // B):
        pltpu.sync_copy(rows[0], acc.at[pl.ds(base + t * B, B)])
    rem = CH - (CH // B) * B
    pltpu.sync_copy(rows[0].at[pl.ds(0, rem)],
                    acc.at[pl.ds(base + (CH // B) * B, rem)])

    @pl.when(s == 0)
    def _ztail():
        pltpu.sync_copy(rows[0].at[pl.ds(0, TAIL)],
                        acc.at[pl.ds(NS * CH, TAIL)])

    plsc.subcore_barrier()

    @pl.when(c == 0)
    def _sc0():
        _edge_pipeline(h_hbm, srcs0_hbm, dsts0_hbm, src_v, dst_ring, rows,
                       acc, rsems, dsems, s, NB0)

    @pl.when(c != 0)
    def _sc1():
        _edge_pipeline(h_hbm, srcs1_hbm, dsts1_hbm, src_v, dst_ring, rows,
                       acc, rsems, dsems, s, NB1)

    plsc.subcore_barrier()
    pltpu.sync_copy(acc.at[pl.ds(s * CH, CH)],
                    out_hbm.at[c, pl.ds(s * CH, CH)])

    @pl.when(s == 0)
    def _wtail():
        pltpu.sync_copy(acc.at[pl.ds(NS * CH, TAIL)],
                        out_hbm.at[c, pl.ds(NS * CH, TAIL)])


def _comb_body(a_ref, b_ref, o_ref):
    o_ref[...] = jnp.maximum(a_ref[0] + b_ref[0], 0.0)


def _combine(a, b):
    m_blk = 1000
    return pl.pallas_call(
        _comb_body,
        grid=(N_NODES // m_blk,),
        in_specs=[
            pl.BlockSpec((1, m_blk, D), lambda i: (0, i, 0)),
            pl.BlockSpec((1, m_blk, D), lambda i: (1, i, 0)),
        ],
        out_specs=pl.BlockSpec((m_blk, D), lambda i: (i, 0)),
        out_shape=jax.ShapeDtypeStruct((N_NODES, D), jnp.float32),
    )(a, b)


@jax.jit
def kernel(x, edge_index, W, b):
    h = _matmul(x, W, b.reshape(1, D))

    src = edge_index[0].astype(jnp.int32)
    dst = edge_index[1].astype(jnp.int32)
    pad = E1_PAD - (N_EDGES - E0)
    srcs0 = src[:E0].reshape(NS, NB0, B)
    dsts0 = dst[:E0].reshape(NS, NB0, B)
    srcs1 = jnp.concatenate(
        [src[E0:], jnp.zeros((pad,), jnp.int32)]).reshape(NS, NB1, B)
    dsts1 = jnp.concatenate(
        [dst[E0:], jnp.full((pad,), DUMMY_DST, jnp.int32)]).reshape(NS, NB1, B)
    acc = _sc_push(h, srcs0, dsts0, srcs1, dsts1)
    return _combine(acc, acc)
